# baseline (speedup 1.0000x reference)
import functools

import jax
import jax.numpy as jnp
from jax import lax
from jax.experimental import pallas as pl
from jax.experimental.pallas import tpu as pltpu

_CENTER_VAR = 0.1
_SIZE_VAR = 0.2
_IOU_T = 0.45
_SCORE_T = 0.01
_TOPK = 100
_NEG = -1e30


def _prologue(lt_ref, bb_ref, pr_ref, probs_ref, box_ref, *, C):
    x = lt_ref[...]
    m = jnp.max(x, axis=0, keepdims=True)
    e = jnp.exp(x - m)
    d = jnp.sum(e, axis=0, keepdims=True)
    probs_ref[...] = e / d

    loc = bb_ref[...]
    p = pr_ref[...]
    cx = loc[0] * _CENTER_VAR * p[2] + p[0]
    cy = loc[1] * _CENTER_VAR * p[3] + p[1]
    w = jnp.exp(loc[2] * _SIZE_VAR) * p[2]
    h = jnp.exp(loc[3] * _SIZE_VAR) * p[3]
    box_ref[...] = jnp.stack(
        [cx - w * 0.5, cy - h * 0.5, cx + w * 0.5, cy + h * 0.5], axis=0
    )


def _perclass(probs_ref, box_ref, out_ref, s_ref, *, N, R, K):
    rows_i = lax.broadcasted_iota(jnp.int32, (R, 128), 0)
    lanes_i = lax.broadcasted_iota(jnp.int32, (R, 128), 1)
    flat = rows_i * 128 + lanes_i

    s_ref[...] = jnp.where(flat < N, probs_ref[0], _NEG)

    li = lax.broadcasted_iota(jnp.int32, (1, 128), 1)
    z = jnp.zeros((1, 128), jnp.float32)

    def body(k, carry):
        ax1, ay1, ax2, ay2, av = carry
        s = s_ref[...]
        m = jnp.max(s)
        idx = jnp.min(jnp.where(s == m, flat, jnp.int32(2**30)))
        r = idx // 128
        l = idx % 128
        lanemask = li == l

        def pick(coord):
            row = box_ref[coord, pl.ds(r, 1), :]
            return jnp.sum(jnp.where(lanemask, row, 0.0))

        x1v = pick(0)
        y1v = pick(1)
        x2v = pick(2)
        y2v = pick(3)

        srow = s_ref[pl.ds(r, 1), :]
        s_ref[pl.ds(r, 1), :] = jnp.where(lanemask, _NEG, srow)

        km = li == k
        ax1 = jnp.where(km, x1v, ax1)
        ay1 = jnp.where(km, y1v, ay1)
        ax2 = jnp.where(km, x2v, ax2)
        ay2 = jnp.where(km, y2v, ay2)
        av = jnp.where(km, m, av)
        return (ax1, ay1, ax2, ay2, av)

    ax1, ay1, ax2, ay2, av = lax.fori_loop(0, K, body, (z, z, z, z, z))

    area = jnp.clip(ax2 - ax1, 0.0, None) * jnp.clip(ay2 - ay1, 0.0, None)

    def nbody(i, keep):
        im = li == i

        def pk(a):
            return jnp.sum(jnp.where(im, a, 0.0))

        x1i = pk(ax1)
        y1i = pk(ay1)
        x2i = pk(ax2)
        y2i = pk(ay2)
        ki = pk(keep)
        ai = pk(area)
        w = jnp.clip(jnp.minimum(x2i, ax2) - jnp.maximum(x1i, ax1), 0.0, None)
        h = jnp.clip(jnp.minimum(y2i, ay2) - jnp.maximum(y1i, ay1), 0.0, None)
        inter = w * h
        iou = inter / (ai + area - inter + 1e-8)
        supp = (iou > _IOU_T) & (ki > 0.0) & (li > i)
        return jnp.where(supp, 0.0, keep)

    keep = lax.fori_loop(0, K, nbody, jnp.ones((1, 128), jnp.float32))
    keepf = keep * (av > _SCORE_T).astype(jnp.float32)

    out = jnp.concatenate(
        [ax1 * keepf, ay1 * keepf, ax2 * keepf, ay2 * keepf, av * keepf,
         jnp.zeros((3, 128), jnp.float32)],
        axis=0,
    )
    out_ref[...] = out.reshape(1, 8, 128)


def kernel(cls_logits, bbox_pred, priors):
    N, C = cls_logits.shape[1], cls_logits.shape[2]
    R = -(-N // 128)
    Np = R * 128

    ltT = jnp.pad(cls_logits[0], ((0, Np - N), (0, 0))).T.reshape(C, R, 128)
    bbT = jnp.pad(bbox_pred[0], ((0, Np - N), (0, 0))).T.reshape(4, R, 128)
    prT = jnp.pad(priors, ((0, Np - N), (0, 0))).T.reshape(4, R, 128)

    RB = R
    for cand in (32, 16, 8, 4, 2):
        if R % cand == 0:
            RB = cand
            break

    probsT, boxes4 = pl.pallas_call(
        functools.partial(_prologue, C=C),
        grid=(R // RB,),
        in_specs=[
            pl.BlockSpec((C, RB, 128), lambda i: (0, i, 0)),
            pl.BlockSpec((4, RB, 128), lambda i: (0, i, 0)),
            pl.BlockSpec((4, RB, 128), lambda i: (0, i, 0)),
        ],
        out_specs=[
            pl.BlockSpec((C, RB, 128), lambda i: (0, i, 0)),
            pl.BlockSpec((4, RB, 128), lambda i: (0, i, 0)),
        ],
        out_shape=[
            jax.ShapeDtypeStruct((C, R, 128), jnp.float32),
            jax.ShapeDtypeStruct((4, R, 128), jnp.float32),
        ],
    )(ltT, bbT, prT)

    out = pl.pallas_call(
        functools.partial(_perclass, N=N, R=R, K=_TOPK),
        grid=(C - 1,),
        in_specs=[
            pl.BlockSpec((1, R, 128), lambda c: (c + 1, 0, 0)),
            pl.BlockSpec((4, R, 128), lambda c: (0, 0, 0)),
        ],
        out_specs=pl.BlockSpec((1, 8, 128), lambda c: (c, 0, 0)),
        out_shape=jax.ShapeDtypeStruct((C - 1, 8, 128), jnp.float32),
        scratch_shapes=[pltpu.VMEM((R, 128), jnp.float32)],
    )(probsT, boxes4)

    return out[:, :5, :_TOPK].transpose(0, 2, 1)
